# baseline (speedup 1.0000x reference)
import functools

import numpy as np
import jax
import jax.numpy as jnp
from jax import lax
from jax.experimental import pallas as pl
from jax.experimental.pallas import tpu as pltpu
from jax.experimental.pallas import tpu_sc as plsc

_BLK = 512


def _pad_up(n, m=_BLK):
    return ((n + m - 1) // m) * m


def _edge_counts(src, dst, n_pad):
    e = src.shape[0]
    info = plsc.get_sparse_core_info()
    nc, ns = info.num_cores, info.num_subcores
    nw = nc * ns
    epw = e // nw
    assert epw * nw == e and epw % 16 == 0 and epw % 8 == 0

    mesh = plsc.VectorSubcoreMesh(core_axis_name="c", subcore_axis_name="s")

    @functools.partial(
        pl.kernel,
        mesh=mesh,
        compiler_params=pltpu.CompilerParams(needs_layout_passes=False),
        out_type=(
            jax.ShapeDtypeStruct((nw, n_pad), jnp.float32),
            jax.ShapeDtypeStruct((nw, n_pad), jnp.float32),
        ),
        scratch_types=[
            pltpu.VMEM((epw,), jnp.int32),
            pltpu.VMEM((epw,), jnp.int32),
            pltpu.VMEM((n_pad,), jnp.float32),
            pltpu.VMEM((n_pad,), jnp.float32),
        ],
    )
    def _k(src_hbm, dst_hbm, deg_out, self_out, sv, dv, hd, hs):
        wid = lax.axis_index("s") * nc + lax.axis_index("c")
        base = wid * epw
        pltpu.sync_copy(src_hbm.at[pl.ds(base, epw)], sv)
        pltpu.sync_copy(dst_hbm.at[pl.ds(base, epw)], dv)

        def zero(i, c):
            hd[pl.ds(i * 16, 16)] = jnp.zeros((16,), jnp.float32)
            hs[pl.ds(i * 16, 16)] = jnp.zeros((16,), jnp.float32)
            return c

        lax.fori_loop(0, n_pad // 16, zero, 0)

        ones = jnp.ones((16,), jnp.float32)

        def body(i, c):
            s = sv[pl.ds(i * 16, 16)]
            d = dv[pl.ds(i * 16, 16)]
            plsc.addupdate_scatter(hd, [d], ones)
            plsc.addupdate_scatter(hs, [d], ones, mask=s == d)
            return c

        lax.fori_loop(0, epw // 16, body, 0)

        pltpu.sync_copy(hd, deg_out.at[wid])
        pltpu.sync_copy(hs, self_out.at[wid])

    dp, sp = _k(src, dst)
    return dp.sum(axis=0), sp.sum(axis=0)


def _feat_body(x_ref, w_ref, s_ref, c_ref, b_ref, y_ref, u_ref):
    y = jnp.dot(x_ref[...], w_ref[...], preferred_element_type=jnp.float32)
    y = y * s_ref[...]
    y_ref[...] = y
    u_ref[...] = c_ref[...] * y + b_ref[...]


def _feat(x, w, s2, c2, b):
    m, d = x.shape
    h = w.shape[1]
    bm = _BLK
    grid = (m // bm,)
    return pl.pallas_call(
        _feat_body,
        grid=grid,
        in_specs=[
            pl.BlockSpec((bm, d), lambda i: (i, 0)),
            pl.BlockSpec((d, h), lambda i: (0, 0)),
            pl.BlockSpec((bm, h), lambda i: (i, 0)),
            pl.BlockSpec((bm, h), lambda i: (i, 0)),
            pl.BlockSpec((1, h), lambda i: (0, 0)),
        ],
        out_specs=[
            pl.BlockSpec((bm, h), lambda i: (i, 0)),
            pl.BlockSpec((bm, h), lambda i: (i, 0)),
        ],
        out_shape=[
            jax.ShapeDtypeStruct((m, h), jnp.float32),
            jax.ShapeDtypeStruct((m, h), jnp.float32),
        ],
    )(x, w, s2, c2, b)


def _mm_tn_body(a_ref, y_ref, s_ref, u_ref, o_ref, *, nk, relu):
    k = pl.program_id(1)
    acc = lax.dot_general(
        a_ref[...].astype(jnp.float32),
        y_ref[...],
        (((0,), (0,)), ((), ())),
        preferred_element_type=jnp.float32,
    )

    @pl.when(k == 0)
    def _():
        o_ref[...] = acc

    @pl.when(k > 0)
    def _():
        o_ref[...] = o_ref[...] + acc

    @pl.when(k == nk - 1)
    def _():
        r = s_ref[...] * o_ref[...] + u_ref[...]
        o_ref[...] = jnp.maximum(r, 0.0) if relu else r


def _mm_tn(a, y, s2, u, relu):
    m = a.shape[0]
    h = y.shape[1]
    bn = bk = _BLK
    nk = m // bk
    grid = (m // bn, nk)
    return pl.pallas_call(
        functools.partial(_mm_tn_body, nk=nk, relu=relu),
        grid=grid,
        in_specs=[
            pl.BlockSpec((bk, bn), lambda j, k: (k, j)),
            pl.BlockSpec((bk, h), lambda j, k: (k, 0)),
            pl.BlockSpec((bn, h), lambda j, k: (j, 0)),
            pl.BlockSpec((bn, h), lambda j, k: (j, 0)),
        ],
        out_specs=pl.BlockSpec((bn, h), lambda j, k: (j, 0)),
        out_shape=jax.ShapeDtypeStruct((m, h), jnp.float32),
    )(a, y, s2, u)


def _mm_nt_body(r_ref, g_ref, o_ref, *, nk, bm):
    i = pl.program_id(0)
    j = pl.program_id(1)
    k = pl.program_id(2)
    acc = lax.dot_general(
        r_ref[...],
        g_ref[...],
        (((1,), (1,)), ((), ())),
        preferred_element_type=jnp.float32,
    )

    @pl.when(k == 0)
    def _():
        o_ref[...] = acc

    @pl.when(k > 0)
    def _():
        o_ref[...] = o_ref[...] + acc

    @pl.when((k == nk - 1) & (i == j))
    def _():
        ri = lax.broadcasted_iota(jnp.int32, (bm, bm), 0)
        ci = lax.broadcasted_iota(jnp.int32, (bm, bm), 1)
        o_ref[...] = jnp.where(ri == ci, 0.0, o_ref[...])


def _mm_nt(r, g):
    m, kk = r.shape
    n = g.shape[0]
    bm = bn = bk = _BLK
    nk = kk // bk
    grid = (m // bm, n // bn, nk)
    return pl.pallas_call(
        functools.partial(_mm_nt_body, nk=nk, bm=bm),
        grid=grid,
        in_specs=[
            pl.BlockSpec((bm, bk), lambda i, j, k: (i, k)),
            pl.BlockSpec((bn, bk), lambda i, j, k: (j, k)),
        ],
        out_specs=pl.BlockSpec((bm, bn), lambda i, j, k: (i, j)),
        out_shape=jax.ShapeDtypeStruct((m, n), jnp.float32),
    )(r, g)


def _gcn_layer(a, xin, w, b, dinv, fill, relu):
    m = xin.shape[0]
    h = w.shape[1]
    s2 = jnp.broadcast_to(dinv[:, None], (m, h))
    c2 = jnp.broadcast_to((dinv * fill)[:, None], (m, h))
    yt, u = _feat(xin, w, s2, c2, b.reshape(1, -1))
    return _mm_tn(a, yt, s2, u, relu)


def _dinv(deg):
    return jnp.where(deg > 0, 1.0 / jnp.sqrt(deg), 0.0)


def kernel(x, edge_index, W0, b0, W1, b1, W2, b2, p1, p2, Wc, bc, Wmu, bmu):
    n, d = x.shape
    h = W0.shape[1]
    e = edge_index.shape[1]
    k1 = int(np.ceil(0.5 * n))
    k2 = int(np.ceil(0.5 * k1))
    np_ = _pad_up(n)
    k1p = _pad_up(k1)
    k2p = _pad_up(k2)
    o = Wmu.shape[1]
    hc = Wc.shape[1]

    src = edge_index[0]
    dst = edge_index[1]

    colsum, d0 = _edge_counts(src, dst, np_)

    A = jnp.zeros((n, n), jnp.float32).at[src, dst].add(1.0)
    ones_e = jnp.ones((e,), jnp.bfloat16)
    A16 = jnp.zeros((np_, np_), jnp.bfloat16).at[src, dst].add(ones_e)
    AT16 = jnp.zeros((np_, np_), jnp.bfloat16).at[dst, src].add(ones_e)

    fill1 = jnp.where(d0[:n] == 0, 2.0, 0.0)
    A_eff = A + jnp.diag(fill1)
    deg1 = colsum[:n] + fill1
    dinv1 = _dinv(deg1)
    An1 = dinv1[:, None] * A_eff * dinv1[None, :]
    h1 = jax.nn.relu(An1.T @ (x @ W0) + b0)

    score1 = jnp.tanh((h1 @ p1) / jnp.linalg.norm(p1))
    vals1, perm1 = lax.top_k(score1, k1)
    hp1 = h1[perm1] * vals1[:, None]

    permp1 = jnp.concatenate(
        [perm1, jnp.full((k1p - k1,), np_ - 1, jnp.int32)]
    )
    iota1 = jnp.arange(k1, dtype=jnp.int32)
    one_b = jnp.ones((k1,), jnp.bfloat16)
    R1 = A16[permp1, :].at[iota1, perm1].set(one_b)
    G1 = AT16[permp1, :].at[iota1, perm1].set(one_b)
    Ap1 = _mm_nt(R1, G1)

    Ap1c = Ap1[:k1, :k1]
    d2 = jnp.diagonal(Ap1c)
    A_eff2 = Ap1c + jnp.diag(jnp.where(d2 == 0, 2.0, 0.0))
    deg2 = A_eff2.sum(axis=0)
    dinv2 = _dinv(deg2)
    An2 = dinv2[:, None] * A_eff2 * dinv2[None, :]
    h2 = jax.nn.relu(An2.T @ (hp1 @ W1) + b1)

    score2 = jnp.tanh((h2 @ p2) / jnp.linalg.norm(p2))
    vals2, perm2 = lax.top_k(score2, k2)

    permp2 = jnp.concatenate(
        [perm2, jnp.full((k2p - k2,), k1p - 1, jnp.int32)]
    )
    iota2 = jnp.arange(k2, dtype=jnp.int32)
    one_b2 = jnp.ones((k2,), jnp.bfloat16)
    Ap1b = Ap1.astype(jnp.bfloat16)
    Ap1bT = Ap1b.T
    R2 = Ap1b[permp2, :].at[iota2, perm2].set(one_b2)
    G2 = Ap1bT[permp2, :].at[iota2, perm2].set(one_b2)
    Ap2 = _mm_nt(R2, G2)
    hp2 = jnp.zeros((k2p, h), jnp.float32).at[:k2].set(
        h2[perm2] * vals2[:, None]
    )

    dinv3 = _dinv(Ap2.sum(axis=0) + 2.0)
    fill3 = jnp.full((k2p,), 2.0, jnp.float32)
    h3 = _gcn_layer(Ap2, hp2, W2, b2, dinv3, fill3, relu=True)

    Ab = (Ap2 != 0).astype(jnp.bfloat16)
    dinv4 = _dinv(jnp.sum(Ap2 != 0, axis=0).astype(jnp.float32) + 1.0)
    fill4 = jnp.ones((k2p,), jnp.float32)
    Wc_p = jnp.zeros((h, h), jnp.float32).at[:, :hc].set(Wc)
    bc_p = jnp.pad(bc, (0, h - hc))
    h4 = _gcn_layer(Ab, h3, Wc_p, bc_p, dinv4, fill4, relu=False)
    Wmu_p = jnp.zeros((h, h), jnp.float32).at[:hc, :o].set(Wmu)
    bmu_p = jnp.pad(bmu, (0, h - o))
    h5 = _gcn_layer(Ab, h4, Wmu_p, bmu_p, dinv4, fill4, relu=False)
    mu = h5[:k2, :o]

    return (mu, mu, perm1, perm2)


# speedup vs baseline: 2.0430x; 2.0430x over previous
import functools

import numpy as np
import jax
import jax.numpy as jnp
from jax import lax
from jax.experimental import pallas as pl
from jax.experimental.pallas import tpu as pltpu
from jax.experimental.pallas import tpu_sc as plsc

_BLK = 512


def _pad_up(n, m=_BLK):
    return ((n + m - 1) // m) * m


def _edge_counts(src, dst, n_pad):
    e = src.shape[0]
    info = plsc.get_sparse_core_info()
    nc, ns = info.num_cores, info.num_subcores
    nw = nc * ns
    epw = e // nw
    assert epw * nw == e and epw % 16 == 0 and epw % 8 == 0

    mesh = plsc.VectorSubcoreMesh(core_axis_name="c", subcore_axis_name="s")

    @functools.partial(
        pl.kernel,
        mesh=mesh,
        compiler_params=pltpu.CompilerParams(needs_layout_passes=False),
        out_type=(
            jax.ShapeDtypeStruct((nw, n_pad), jnp.float32),
            jax.ShapeDtypeStruct((nw, n_pad), jnp.float32),
        ),
        scratch_types=[
            pltpu.VMEM((epw,), jnp.int32),
            pltpu.VMEM((epw,), jnp.int32),
            pltpu.VMEM((n_pad,), jnp.float32),
            pltpu.VMEM((n_pad,), jnp.float32),
        ],
    )
    def _k(src_hbm, dst_hbm, deg_out, self_out, sv, dv, hd, hs):
        wid = lax.axis_index("s") * nc + lax.axis_index("c")
        base = wid * epw
        pltpu.sync_copy(src_hbm.at[pl.ds(base, epw)], sv)
        pltpu.sync_copy(dst_hbm.at[pl.ds(base, epw)], dv)

        def zero(i, c):
            hd[pl.ds(i * 16, 16)] = jnp.zeros((16,), jnp.float32)
            hs[pl.ds(i * 16, 16)] = jnp.zeros((16,), jnp.float32)
            return c

        lax.fori_loop(0, n_pad // 16, zero, 0)

        ones = jnp.ones((16,), jnp.float32)

        def body(i, c):
            s = sv[pl.ds(i * 16, 16)]
            d = dv[pl.ds(i * 16, 16)]
            plsc.addupdate_scatter(hd, [d], ones)
            plsc.addupdate_scatter(hs, [d], ones, mask=s == d)
            return c

        lax.fori_loop(0, epw // 16, body, 0)

        pltpu.sync_copy(hd, deg_out.at[wid])
        pltpu.sync_copy(hs, self_out.at[wid])

    dp, sp = _k(src, dst)
    return dp.sum(axis=0), sp.sum(axis=0)


def _feat_body(x_ref, w_ref, s_ref, c_ref, b_ref, y_ref, u_ref):
    y = jnp.dot(x_ref[...], w_ref[...], preferred_element_type=jnp.float32)
    y = y * s_ref[...]
    y_ref[...] = y
    u_ref[...] = c_ref[...] * y + b_ref[...]


def _feat(x, w, s2, c2, b):
    m, d = x.shape
    h = w.shape[1]
    bm = _BLK
    grid = (m // bm,)
    return pl.pallas_call(
        _feat_body,
        grid=grid,
        in_specs=[
            pl.BlockSpec((bm, d), lambda i: (i, 0)),
            pl.BlockSpec((d, h), lambda i: (0, 0)),
            pl.BlockSpec((bm, h), lambda i: (i, 0)),
            pl.BlockSpec((bm, h), lambda i: (i, 0)),
            pl.BlockSpec((1, h), lambda i: (0, 0)),
        ],
        out_specs=[
            pl.BlockSpec((bm, h), lambda i: (i, 0)),
            pl.BlockSpec((bm, h), lambda i: (i, 0)),
        ],
        out_shape=[
            jax.ShapeDtypeStruct((m, h), jnp.float32),
            jax.ShapeDtypeStruct((m, h), jnp.float32),
        ],
    )(x, w, s2, c2, b)


def _mm_tn_body(a_ref, y_ref, s_ref, u_ref, o_ref, *, nk, relu):
    k = pl.program_id(1)
    acc = lax.dot_general(
        a_ref[...].astype(jnp.float32),
        y_ref[...],
        (((0,), (0,)), ((), ())),
        preferred_element_type=jnp.float32,
    )

    @pl.when(k == 0)
    def _():
        o_ref[...] = acc

    @pl.when(k > 0)
    def _():
        o_ref[...] = o_ref[...] + acc

    @pl.when(k == nk - 1)
    def _():
        r = s_ref[...] * o_ref[...] + u_ref[...]
        o_ref[...] = jnp.maximum(r, 0.0) if relu else r


def _mm_tn(a, y, s2, u, relu):
    m = a.shape[0]
    h = y.shape[1]
    bn = bk = _BLK
    nk = m // bk
    grid = (m // bn, nk)
    return pl.pallas_call(
        functools.partial(_mm_tn_body, nk=nk, relu=relu),
        grid=grid,
        in_specs=[
            pl.BlockSpec((bk, bn), lambda j, k: (k, j)),
            pl.BlockSpec((bk, h), lambda j, k: (k, 0)),
            pl.BlockSpec((bn, h), lambda j, k: (j, 0)),
            pl.BlockSpec((bn, h), lambda j, k: (j, 0)),
        ],
        out_specs=pl.BlockSpec((bn, h), lambda j, k: (j, 0)),
        out_shape=jax.ShapeDtypeStruct((m, h), jnp.float32),
    )(a, y, s2, u)


def _mm_nt_body(r_ref, g_ref, o_ref, *, nk, bm, valid):
    i = pl.program_id(0)
    j = pl.program_id(1)
    k = pl.program_id(2)
    acc = lax.dot_general(
        r_ref[...].astype(jnp.bfloat16),
        g_ref[...].astype(jnp.bfloat16),
        (((1,), (1,)), ((), ())),
        preferred_element_type=jnp.float32,
    )

    @pl.when(k == 0)
    def _():
        o_ref[...] = acc

    @pl.when(k > 0)
    def _():
        o_ref[...] = o_ref[...] + acc

    @pl.when(k == nk - 1)
    def _():
        ri = i * bm + lax.broadcasted_iota(jnp.int32, (bm, bm), 0)
        ci = j * bm + lax.broadcasted_iota(jnp.int32, (bm, bm), 1)
        keep = (ri != ci) & (ri < valid) & (ci < valid)
        o_ref[...] = jnp.where(keep, o_ref[...], 0.0)


def _mm_nt(r, g, valid):
    m, kk = r.shape
    n = g.shape[0]
    bm = bn = 1024 if (m % 1024 == 0 and n % 1024 == 0) else _BLK
    bk = _BLK
    nk = kk // bk
    grid = (m // bm, n // bn, nk)
    return pl.pallas_call(
        functools.partial(_mm_nt_body, nk=nk, bm=bm, valid=valid),
        grid=grid,
        in_specs=[
            pl.BlockSpec((bm, bk), lambda i, j, k: (i, k)),
            pl.BlockSpec((bn, bk), lambda i, j, k: (j, k)),
        ],
        out_specs=pl.BlockSpec((bm, bn), lambda i, j, k: (i, j)),
        out_shape=jax.ShapeDtypeStruct((m, n), jnp.float32),
    )(r, g)


def _gcn_layer(a, xin, w, b, dinv, fill, relu):
    m = xin.shape[0]
    h = w.shape[1]
    s2 = jnp.broadcast_to(dinv[:, None], (m, h))
    c2 = jnp.broadcast_to((dinv * fill)[:, None], (m, h))
    yt, u = _feat(xin, w, s2, c2, b.reshape(1, -1))
    return _mm_tn(a, yt, s2, u, relu)


def _dinv(deg):
    return jnp.where(deg > 0, 1.0 / jnp.sqrt(deg), 0.0)


def kernel(x, edge_index, W0, b0, W1, b1, W2, b2, p1, p2, Wc, bc, Wmu, bmu):
    n, d = x.shape
    h = W0.shape[1]
    e = edge_index.shape[1]
    k1 = int(np.ceil(0.5 * n))
    k2 = int(np.ceil(0.5 * k1))
    np_ = _pad_up(n)
    k1p = _pad_up(k1)
    k2p = _pad_up(k2)
    o = Wmu.shape[1]
    hc = Wc.shape[1]

    src = edge_index[0]
    dst = edge_index[1]

    colsum, d0 = _edge_counts(src, dst, np_)

    A10 = jnp.zeros((np_, np_), jnp.float32).at[src, dst].add(1.0)
    A = A10[:n, :n]
    diag_i = jnp.arange(n, dtype=jnp.int32)
    Alf = A10.at[diag_i, diag_i].set(1.0)
    AlfT = Alf.T

    fill1 = jnp.where(d0[:n] == 0, 2.0, 0.0)
    A_eff = A + jnp.diag(fill1)
    deg1 = colsum[:n] + fill1
    dinv1 = _dinv(deg1)
    An1 = dinv1[:, None] * A_eff * dinv1[None, :]
    h1 = jax.nn.relu(An1.T @ (x @ W0) + b0)

    score1 = jnp.tanh((h1 @ p1) / jnp.linalg.norm(p1))
    vals1, perm1 = lax.top_k(score1, k1)
    hp1 = h1[perm1] * vals1[:, None]

    permp1 = jnp.pad(perm1, (0, k1p - k1))
    R1 = Alf[permp1, :]
    G1 = AlfT[permp1, :]
    Ap1 = _mm_nt(R1, G1, k1)

    Ap1c = Ap1[:k1, :k1]
    d2 = jnp.diagonal(Ap1c)
    A_eff2 = Ap1c + jnp.diag(jnp.where(d2 == 0, 2.0, 0.0))
    deg2 = A_eff2.sum(axis=0)
    dinv2 = _dinv(deg2)
    An2 = dinv2[:, None] * A_eff2 * dinv2[None, :]
    h2 = jax.nn.relu(An2.T @ (hp1 @ W1) + b1)

    score2 = jnp.tanh((h2 @ p2) / jnp.linalg.norm(p2))
    vals2, perm2 = lax.top_k(score2, k2)

    permp2 = jnp.pad(perm2, (0, k2p - k2))
    diag_i2 = jnp.arange(k1, dtype=jnp.int32)
    Ap1l = Ap1.at[diag_i2, diag_i2].set(1.0)
    Ap1lT = Ap1l.T
    R2 = Ap1l[permp2, :]
    G2 = Ap1lT[permp2, :]
    Ap2 = _mm_nt(R2, G2, k2)
    hp2 = jnp.zeros((k2p, h), jnp.float32).at[:k2].set(
        h2[perm2] * vals2[:, None]
    )

    dinv3 = _dinv(Ap2.sum(axis=0) + 2.0)
    fill3 = jnp.full((k2p,), 2.0, jnp.float32)
    h3 = _gcn_layer(Ap2, hp2, W2, b2, dinv3, fill3, relu=True)

    Ab = (Ap2 != 0).astype(jnp.bfloat16)
    dinv4 = _dinv(jnp.sum(Ap2 != 0, axis=0).astype(jnp.float32) + 1.0)
    fill4 = jnp.ones((k2p,), jnp.float32)
    Wc_p = jnp.zeros((h, h), jnp.float32).at[:, :hc].set(Wc)
    bc_p = jnp.pad(bc, (0, h - hc))
    h4 = _gcn_layer(Ab, h3, Wc_p, bc_p, dinv4, fill4, relu=False)
    Wmu_p = jnp.zeros((h, h), jnp.float32).at[:hc, :o].set(Wmu)
    bmu_p = jnp.pad(bmu, (0, h - o))
    h5 = _gcn_layer(Ab, h4, Wmu_p, bmu_p, dinv4, fill4, relu=False)
    mu = h5[:k2, :o]

    return (mu, mu, perm1, perm2)


# speedup vs baseline: 2.1470x; 1.0509x over previous
import functools

import numpy as np
import jax
import jax.numpy as jnp
from jax import lax
from jax.experimental import pallas as pl
from jax.experimental.pallas import tpu as pltpu
from jax.experimental.pallas import tpu_sc as plsc

_BLK = 512


def _pad_up(n, m=_BLK):
    return ((n + m - 1) // m) * m


def _edge_counts(src, dst, n_pad):
    e = src.shape[0]
    info = plsc.get_sparse_core_info()
    nc, ns = info.num_cores, info.num_subcores
    nw = nc * ns
    epw = e // nw
    assert epw * nw == e and epw % 16 == 0 and epw % 8 == 0

    mesh = plsc.VectorSubcoreMesh(core_axis_name="c", subcore_axis_name="s")

    @functools.partial(
        pl.kernel,
        mesh=mesh,
        compiler_params=pltpu.CompilerParams(needs_layout_passes=False),
        out_type=(
            jax.ShapeDtypeStruct((nw, n_pad), jnp.float32),
            jax.ShapeDtypeStruct((nw, n_pad), jnp.float32),
        ),
        scratch_types=[
            pltpu.VMEM((epw,), jnp.int32),
            pltpu.VMEM((epw,), jnp.int32),
            pltpu.VMEM((n_pad,), jnp.float32),
            pltpu.VMEM((n_pad,), jnp.float32),
        ],
    )
    def _k(src_hbm, dst_hbm, deg_out, self_out, sv, dv, hd, hs):
        wid = lax.axis_index("s") * nc + lax.axis_index("c")
        base = wid * epw
        pltpu.sync_copy(src_hbm.at[pl.ds(base, epw)], sv)
        pltpu.sync_copy(dst_hbm.at[pl.ds(base, epw)], dv)

        def zero(i, c):
            hd[pl.ds(i * 16, 16)] = jnp.zeros((16,), jnp.float32)
            hs[pl.ds(i * 16, 16)] = jnp.zeros((16,), jnp.float32)
            return c

        lax.fori_loop(0, n_pad // 16, zero, 0)

        ones = jnp.ones((16,), jnp.float32)

        def body(i, c):
            s = sv[pl.ds(i * 16, 16)]
            d = dv[pl.ds(i * 16, 16)]
            plsc.addupdate_scatter(hd, [d], ones)
            plsc.addupdate_scatter(hs, [d], ones, mask=s == d)
            return c

        lax.fori_loop(0, epw // 16, body, 0)

        pltpu.sync_copy(hd, deg_out.at[wid])
        pltpu.sync_copy(hs, self_out.at[wid])

    dp, sp = _k(src, dst)
    return dp.sum(axis=0), sp.sum(axis=0)


def _feat_body(x_ref, w_ref, s_ref, c_ref, b_ref, y_ref, u_ref):
    y = jnp.dot(x_ref[...], w_ref[...], preferred_element_type=jnp.float32)
    y = y * s_ref[...]
    y_ref[...] = y
    u_ref[...] = c_ref[...] * y + b_ref[...]


def _feat(x, w, s2, c2, b):
    m, d = x.shape
    h = w.shape[1]
    bm = _BLK
    grid = (m // bm,)
    return pl.pallas_call(
        _feat_body,
        grid=grid,
        in_specs=[
            pl.BlockSpec((bm, d), lambda i: (i, 0)),
            pl.BlockSpec((d, h), lambda i: (0, 0)),
            pl.BlockSpec((bm, h), lambda i: (i, 0)),
            pl.BlockSpec((bm, h), lambda i: (i, 0)),
            pl.BlockSpec((1, h), lambda i: (0, 0)),
        ],
        out_specs=[
            pl.BlockSpec((bm, h), lambda i: (i, 0)),
            pl.BlockSpec((bm, h), lambda i: (i, 0)),
        ],
        out_shape=[
            jax.ShapeDtypeStruct((m, h), jnp.float32),
            jax.ShapeDtypeStruct((m, h), jnp.float32),
        ],
    )(x, w, s2, c2, b)


def _mm_tn_body(a_ref, y_ref, s_ref, u_ref, o_ref, *, nk, relu):
    k = pl.program_id(1)
    acc = lax.dot_general(
        a_ref[...].astype(jnp.float32),
        y_ref[...],
        (((0,), (0,)), ((), ())),
        preferred_element_type=jnp.float32,
    )

    @pl.when(k == 0)
    def _():
        o_ref[...] = acc

    @pl.when(k > 0)
    def _():
        o_ref[...] = o_ref[...] + acc

    @pl.when(k == nk - 1)
    def _():
        r = s_ref[...] * o_ref[...] + u_ref[...]
        o_ref[...] = jnp.maximum(r, 0.0) if relu else r


def _mm_tn(a, y, s2, u, relu):
    m = a.shape[0]
    h = y.shape[1]
    bn = bk = _BLK
    nk = m // bk
    grid = (m // bn, nk)
    return pl.pallas_call(
        functools.partial(_mm_tn_body, nk=nk, relu=relu),
        grid=grid,
        in_specs=[
            pl.BlockSpec((bk, bn), lambda j, k: (k, j)),
            pl.BlockSpec((bk, h), lambda j, k: (k, 0)),
            pl.BlockSpec((bn, h), lambda j, k: (j, 0)),
            pl.BlockSpec((bn, h), lambda j, k: (j, 0)),
        ],
        out_specs=pl.BlockSpec((bn, h), lambda j, k: (j, 0)),
        out_shape=jax.ShapeDtypeStruct((m, h), jnp.float32),
    )(a, y, s2, u)


def _mm_nt_body(r_ref, g_ref, o_ref, *, nk, bm, valid):
    i = pl.program_id(0)
    j = pl.program_id(1)
    k = pl.program_id(2)
    acc = lax.dot_general(
        r_ref[...].astype(jnp.bfloat16),
        g_ref[...].astype(jnp.bfloat16),
        (((1,), (1,)), ((), ())),
        preferred_element_type=jnp.float32,
    )

    @pl.when(k == 0)
    def _():
        o_ref[...] = acc

    @pl.when(k > 0)
    def _():
        o_ref[...] = o_ref[...] + acc

    @pl.when(k == nk - 1)
    def _():
        ri = i * bm + lax.broadcasted_iota(jnp.int32, (bm, bm), 0)
        ci = j * bm + lax.broadcasted_iota(jnp.int32, (bm, bm), 1)
        keep = (ri != ci) & (ri < valid) & (ci < valid)
        o_ref[...] = jnp.where(keep, o_ref[...], 0.0)


def _mm_nt(r, g, valid):
    m, kk = r.shape
    n = g.shape[0]
    bm = bn = 1024 if (m % 1024 == 0 and n % 1024 == 0) else _BLK
    bk = 1024 if kk % 1024 == 0 else _BLK
    nk = kk // bk
    grid = (m // bm, n // bn, nk)
    return pl.pallas_call(
        functools.partial(_mm_nt_body, nk=nk, bm=bm, valid=valid),
        grid=grid,
        in_specs=[
            pl.BlockSpec((bm, bk), lambda i, j, k: (i, k)),
            pl.BlockSpec((bn, bk), lambda i, j, k: (j, k)),
        ],
        out_specs=pl.BlockSpec((bm, bn), lambda i, j, k: (i, j)),
        out_shape=jax.ShapeDtypeStruct((m, n), jnp.float32),
    )(r, g)


def _gcn_layer(a, xin, w, b, dinv, fill, relu):
    m = xin.shape[0]
    h = w.shape[1]
    s2 = jnp.broadcast_to(dinv[:, None], (m, h))
    c2 = jnp.broadcast_to((dinv * fill)[:, None], (m, h))
    yt, u = _feat(xin, w, s2, c2, b.reshape(1, -1))
    return _mm_tn(a, yt, s2, u, relu)


def _dinv(deg):
    return jnp.where(deg > 0, 1.0 / jnp.sqrt(deg), 0.0)


def kernel(x, edge_index, W0, b0, W1, b1, W2, b2, p1, p2, Wc, bc, Wmu, bmu):
    n, d = x.shape
    h = W0.shape[1]
    e = edge_index.shape[1]
    k1 = int(np.ceil(0.5 * n))
    k2 = int(np.ceil(0.5 * k1))
    np_ = _pad_up(n)
    k1p = _pad_up(k1)
    k2p = _pad_up(k2)
    o = Wmu.shape[1]
    hc = Wc.shape[1]

    src = edge_index[0]
    dst = edge_index[1]

    colsum, d0 = _edge_counts(src, dst, np_)

    A10 = jnp.zeros((np_, np_), jnp.float32).at[src, dst].add(1.0)
    A = A10[:n, :n]
    diag_i = jnp.arange(n, dtype=jnp.int32)
    Alf = A10.at[diag_i, diag_i].set(1.0)
    AlfT = Alf.T

    fill1 = jnp.where(d0[:n] == 0, 2.0, 0.0)
    A_eff = A + jnp.diag(fill1)
    deg1 = colsum[:n] + fill1
    dinv1 = _dinv(deg1)
    An1 = dinv1[:, None] * A_eff * dinv1[None, :]
    h1 = jax.nn.relu(An1.T @ (x @ W0) + b0)

    score1 = jnp.tanh((h1 @ p1) / jnp.linalg.norm(p1))
    vals1, perm1 = lax.top_k(score1, k1)
    hp1 = h1[perm1] * vals1[:, None]

    permp1 = jnp.pad(perm1, (0, k1p - k1))
    R1 = Alf[permp1, :]
    G1 = AlfT[permp1, :]
    Ap1 = _mm_nt(R1, G1, k1)

    Ap1c = Ap1[:k1, :k1]
    d2 = jnp.diagonal(Ap1c)
    A_eff2 = Ap1c + jnp.diag(jnp.where(d2 == 0, 2.0, 0.0))
    deg2 = A_eff2.sum(axis=0)
    dinv2 = _dinv(deg2)
    An2 = dinv2[:, None] * A_eff2 * dinv2[None, :]
    h2 = jax.nn.relu(An2.T @ (hp1 @ W1) + b1)

    score2 = jnp.tanh((h2 @ p2) / jnp.linalg.norm(p2))
    vals2, perm2 = lax.top_k(score2, k2)

    permp2 = jnp.pad(perm2, (0, k2p - k2))
    diag_i2 = jnp.arange(k1, dtype=jnp.int32)
    Ap1l = Ap1.at[diag_i2, diag_i2].set(1.0)
    Ap1lT = Ap1l.T
    R2 = Ap1l[permp2, :]
    G2 = Ap1lT[permp2, :]
    Ap2 = _mm_nt(R2, G2, k2)
    hp2 = jnp.zeros((k2p, h), jnp.float32).at[:k2].set(
        h2[perm2] * vals2[:, None]
    )

    dinv3 = _dinv(Ap2.sum(axis=0) + 2.0)
    fill3 = jnp.full((k2p,), 2.0, jnp.float32)
    h3 = _gcn_layer(Ap2, hp2, W2, b2, dinv3, fill3, relu=True)

    Ab = (Ap2 != 0).astype(jnp.bfloat16)
    dinv4 = _dinv(jnp.sum(Ap2 != 0, axis=0).astype(jnp.float32) + 1.0)
    fill4 = jnp.ones((k2p,), jnp.float32)
    Wc_p = jnp.zeros((h, h), jnp.float32).at[:, :hc].set(Wc)
    bc_p = jnp.pad(bc, (0, h - hc))
    h4 = _gcn_layer(Ab, h3, Wc_p, bc_p, dinv4, fill4, relu=False)
    Wmu_p = jnp.zeros((h, h), jnp.float32).at[:hc, :o].set(Wmu)
    bmu_p = jnp.pad(bmu, (0, h - o))
    h5 = _gcn_layer(Ab, h4, Wmu_p, bmu_p, dinv4, fill4, relu=False)
    mu = h5[:k2, :o]

    return (mu, mu, perm1, perm2)


# speedup vs baseline: 2.2455x; 1.0459x over previous
import functools

import numpy as np
import jax
import jax.numpy as jnp
from jax import lax
from jax.experimental import pallas as pl
from jax.experimental.pallas import tpu as pltpu
from jax.experimental.pallas import tpu_sc as plsc

_BLK = 512


def _pad_up(n, m=_BLK):
    return ((n + m - 1) // m) * m


def _edge_counts(src, dst, n_pad):
    e = src.shape[0]
    info = plsc.get_sparse_core_info()
    nc, ns = info.num_cores, info.num_subcores
    nw = nc * ns
    epw = e // nw
    assert epw * nw == e and epw % 16 == 0 and epw % 8 == 0

    mesh = plsc.VectorSubcoreMesh(core_axis_name="c", subcore_axis_name="s")

    @functools.partial(
        pl.kernel,
        mesh=mesh,
        compiler_params=pltpu.CompilerParams(needs_layout_passes=False),
        out_type=(
            jax.ShapeDtypeStruct((nw, n_pad), jnp.float32),
            jax.ShapeDtypeStruct((nw, n_pad), jnp.float32),
        ),
        scratch_types=[
            pltpu.VMEM((epw,), jnp.int32),
            pltpu.VMEM((epw,), jnp.int32),
            pltpu.VMEM((n_pad,), jnp.float32),
            pltpu.VMEM((n_pad,), jnp.float32),
        ],
    )
    def _k(src_hbm, dst_hbm, deg_out, self_out, sv, dv, hd, hs):
        wid = lax.axis_index("s") * nc + lax.axis_index("c")
        base = wid * epw
        pltpu.sync_copy(src_hbm.at[pl.ds(base, epw)], sv)
        pltpu.sync_copy(dst_hbm.at[pl.ds(base, epw)], dv)

        def zero(i, c):
            hd[pl.ds(i * 16, 16)] = jnp.zeros((16,), jnp.float32)
            hs[pl.ds(i * 16, 16)] = jnp.zeros((16,), jnp.float32)
            return c

        lax.fori_loop(0, n_pad // 16, zero, 0)

        ones = jnp.ones((16,), jnp.float32)

        def body(i, c):
            s = sv[pl.ds(i * 16, 16)]
            d = dv[pl.ds(i * 16, 16)]
            plsc.addupdate_scatter(hd, [d], ones)
            plsc.addupdate_scatter(hs, [d], ones, mask=s == d)
            return c

        lax.fori_loop(0, epw // 16, body, 0)

        pltpu.sync_copy(hd, deg_out.at[wid])
        pltpu.sync_copy(hs, self_out.at[wid])

    dp, sp = _k(src, dst)
    return dp.sum(axis=0), sp.sum(axis=0)


def _feat_body(x_ref, w_ref, s_ref, c_ref, b_ref, y_ref, u_ref):
    y = jnp.dot(x_ref[...], w_ref[...], preferred_element_type=jnp.float32)
    y = y * s_ref[...]
    y_ref[...] = y
    u_ref[...] = c_ref[...] * y + b_ref[...]


def _feat(x, w, s2, c2, b):
    m, d = x.shape
    h = w.shape[1]
    bm = _BLK
    grid = (m // bm,)
    return pl.pallas_call(
        _feat_body,
        grid=grid,
        in_specs=[
            pl.BlockSpec((bm, d), lambda i: (i, 0)),
            pl.BlockSpec((d, h), lambda i: (0, 0)),
            pl.BlockSpec((bm, h), lambda i: (i, 0)),
            pl.BlockSpec((bm, h), lambda i: (i, 0)),
            pl.BlockSpec((1, h), lambda i: (0, 0)),
        ],
        out_specs=[
            pl.BlockSpec((bm, h), lambda i: (i, 0)),
            pl.BlockSpec((bm, h), lambda i: (i, 0)),
        ],
        out_shape=[
            jax.ShapeDtypeStruct((m, h), jnp.float32),
            jax.ShapeDtypeStruct((m, h), jnp.float32),
        ],
    )(x, w, s2, c2, b)


def _mm_tn_body(a_ref, y_ref, s_ref, u_ref, o_ref, *, nk, relu):
    k = pl.program_id(1)
    acc = lax.dot_general(
        a_ref[...].astype(jnp.float32),
        y_ref[...],
        (((0,), (0,)), ((), ())),
        preferred_element_type=jnp.float32,
    )

    @pl.when(k == 0)
    def _():
        o_ref[...] = acc

    @pl.when(k > 0)
    def _():
        o_ref[...] = o_ref[...] + acc

    @pl.when(k == nk - 1)
    def _():
        r = s_ref[...] * o_ref[...] + u_ref[...]
        o_ref[...] = jnp.maximum(r, 0.0) if relu else r


def _mm_tn(a, y, s2, u, relu):
    m = a.shape[0]
    h = y.shape[1]
    bn = bk = _BLK
    nk = m // bk
    grid = (m // bn, nk)
    return pl.pallas_call(
        functools.partial(_mm_tn_body, nk=nk, relu=relu),
        grid=grid,
        in_specs=[
            pl.BlockSpec((bk, bn), lambda j, k: (k, j)),
            pl.BlockSpec((bk, h), lambda j, k: (k, 0)),
            pl.BlockSpec((bn, h), lambda j, k: (j, 0)),
            pl.BlockSpec((bn, h), lambda j, k: (j, 0)),
        ],
        out_specs=pl.BlockSpec((bn, h), lambda j, k: (j, 0)),
        out_shape=jax.ShapeDtypeStruct((m, h), jnp.float32),
    )(a, y, s2, u)


def _mm_nt_body(r_ref, g_ref, o_ref, *, nk, bm, valid):
    i = pl.program_id(0)
    j = pl.program_id(1)
    k = pl.program_id(2)
    acc = lax.dot_general(
        r_ref[...].astype(jnp.bfloat16),
        g_ref[...].astype(jnp.bfloat16),
        (((1,), (1,)), ((), ())),
        preferred_element_type=jnp.float32,
    )

    @pl.when(k == 0)
    def _():
        o_ref[...] = acc

    @pl.when(k > 0)
    def _():
        o_ref[...] = o_ref[...] + acc

    @pl.when(k == nk - 1)
    def _():
        ri = i * bm + lax.broadcasted_iota(jnp.int32, (bm, bm), 0)
        ci = j * bm + lax.broadcasted_iota(jnp.int32, (bm, bm), 1)
        keep = (ri != ci) & (ri < valid) & (ci < valid)
        o_ref[...] = jnp.where(keep, o_ref[...], 0.0)


def _mm_nt(r, g, valid):
    m, kk = r.shape
    n = g.shape[0]
    bm = bn = 1024 if (m % 1024 == 0 and n % 1024 == 0) else _BLK
    bk = 1024 if kk % 1024 == 0 else _BLK
    nk = kk // bk
    grid = (m // bm, n // bn, nk)
    return pl.pallas_call(
        functools.partial(_mm_nt_body, nk=nk, bm=bm, valid=valid),
        grid=grid,
        in_specs=[
            pl.BlockSpec((bm, bk), lambda i, j, k: (i, k)),
            pl.BlockSpec((bn, bk), lambda i, j, k: (j, k)),
        ],
        out_specs=pl.BlockSpec((bm, bn), lambda i, j, k: (i, j)),
        out_shape=jax.ShapeDtypeStruct((m, n), jnp.float32),
    )(r, g)


def _gcn_layer(a, xin, w, b, dinv, fill, relu):
    m = xin.shape[0]
    h = w.shape[1]
    s2 = jnp.broadcast_to(dinv[:, None], (m, h))
    c2 = jnp.broadcast_to((dinv * fill)[:, None], (m, h))
    yt, u = _feat(xin, w, s2, c2, b.reshape(1, -1))
    return _mm_tn(a, yt, s2, u, relu)


def _dinv(deg):
    return jnp.where(deg > 0, 1.0 / jnp.sqrt(deg), 0.0)


def kernel(x, edge_index, W0, b0, W1, b1, W2, b2, p1, p2, Wc, bc, Wmu, bmu):
    n, d = x.shape
    h = W0.shape[1]
    e = edge_index.shape[1]
    k1 = int(np.ceil(0.5 * n))
    k2 = int(np.ceil(0.5 * k1))
    np_ = _pad_up(n)
    k1p = _pad_up(k1)
    k2p = _pad_up(k2)
    o = Wmu.shape[1]
    hc = Wc.shape[1]

    src = edge_index[0]
    dst = edge_index[1]

    colsum, d0 = _edge_counts(src, dst, np_)

    A10 = jnp.zeros((np_, np_), jnp.float32).at[src, dst].add(1.0)
    A = A10[:n, :n]
    diag_i = jnp.arange(n, dtype=jnp.int32)
    Alf = A10.at[diag_i, diag_i].set(1.0).astype(jnp.bfloat16)
    AlfT = Alf.T

    fill1 = jnp.where(d0[:n] == 0, 2.0, 0.0)
    A_eff = A + jnp.diag(fill1)
    deg1 = colsum[:n] + fill1
    dinv1 = _dinv(deg1)
    An1 = dinv1[:, None] * A_eff * dinv1[None, :]
    h1 = jax.nn.relu(An1.T @ (x @ W0) + b0)

    score1 = jnp.tanh((h1 @ p1) / jnp.linalg.norm(p1))
    vals1, perm1 = lax.top_k(score1, k1)
    hp1 = h1[perm1] * vals1[:, None]

    permp1 = jnp.pad(perm1, (0, k1p - k1))
    R1 = Alf[permp1, :]
    G1 = AlfT[permp1, :]
    Ap1 = _mm_nt(R1, G1, k1)

    Ap1c = Ap1[:k1, :k1]
    d2 = jnp.diagonal(Ap1c)
    A_eff2 = Ap1c + jnp.diag(jnp.where(d2 == 0, 2.0, 0.0))
    deg2 = A_eff2.sum(axis=0)
    dinv2 = _dinv(deg2)
    An2 = dinv2[:, None] * A_eff2 * dinv2[None, :]
    h2 = jax.nn.relu(An2.T @ (hp1 @ W1) + b1)

    score2 = jnp.tanh((h2 @ p2) / jnp.linalg.norm(p2))
    vals2, perm2 = lax.top_k(score2, k2)

    permp2 = jnp.pad(perm2, (0, k2p - k2))
    diag_i2 = jnp.arange(k1, dtype=jnp.int32)
    Ap1l = Ap1.at[diag_i2, diag_i2].set(1.0)
    Ap1lT = Ap1l.T
    R2 = Ap1l[permp2, :]
    G2 = Ap1lT[permp2, :]
    Ap2 = _mm_nt(R2, G2, k2)
    hp2 = jnp.zeros((k2p, h), jnp.float32).at[:k2].set(
        h2[perm2] * vals2[:, None]
    )

    dinv3 = _dinv(Ap2.sum(axis=0) + 2.0)
    fill3 = jnp.full((k2p,), 2.0, jnp.float32)
    h3 = _gcn_layer(Ap2, hp2, W2, b2, dinv3, fill3, relu=True)

    Ab = (Ap2 != 0).astype(jnp.bfloat16)
    dinv4 = _dinv(jnp.sum(Ap2 != 0, axis=0).astype(jnp.float32) + 1.0)
    fill4 = jnp.ones((k2p,), jnp.float32)
    Wc_p = jnp.zeros((h, h), jnp.float32).at[:, :hc].set(Wc)
    bc_p = jnp.pad(bc, (0, h - hc))
    h4 = _gcn_layer(Ab, h3, Wc_p, bc_p, dinv4, fill4, relu=False)
    Wmu_p = jnp.zeros((h, h), jnp.float32).at[:hc, :o].set(Wmu)
    bmu_p = jnp.pad(bmu, (0, h - o))
    h5 = _gcn_layer(Ab, h4, Wmu_p, bmu_p, dinv4, fill4, relu=False)
    mu = h5[:k2, :o]

    return (mu, mu, perm1, perm2)
